# baseline (speedup 1.0000x reference)
import functools

import jax
import jax.numpy as jnp
from jax import lax
from jax.experimental import pallas as pl
from jax.experimental.pallas import tpu as pltpu
from jax.experimental.pallas import tpu_sc as plsc

NUM_CLASS = 19
NBINS = NUM_CLASS * NUM_CLASS
BINS_PAD = 368
LANES = 16

N_TOTAL = 16 * 512 * 512
NC = 2
NS = 16
NW = NC * NS
IMG_H = 512
IMG_W = 512
ROWS_PER_W = 256
ROWS_PER_CHUNK = 32
CHUNK = ROWS_PER_CHUNK * IMG_W
N_CHUNKS = ROWS_PER_W // ROWS_PER_CHUNK
VECS_PER_CHUNK = CHUNK // LANES
VECS_PER_ROW = IMG_W // LANES


def _sc_body(gt_hbm, pre_hbm, out_hbm, gt_buf, pre_buf, hist, hist1d,
             sem_g, sem_p):
  wid = lax.axis_index("s") * NC + lax.axis_index("c")
  img = wid // 2
  row_base = (wid % 2) * ROWS_PER_W

  lane = jnp.arange(LANES, dtype=jnp.int32)
  ones = jnp.ones((LANES,), jnp.float32)
  zeros = jnp.zeros((LANES,), jnp.float32)

  def issue(c, b):
    r0 = row_base + c * ROWS_PER_CHUNK
    def row_copy(r, _):
      dst = pl.ds(b * CHUNK + r * IMG_W, IMG_W)
      pltpu.async_copy(gt_hbm.at[img, r0 + r, :], gt_buf.at[dst], sem_g.at[b])
      pltpu.async_copy(pre_hbm.at[img, r0 + r, :], pre_buf.at[dst], sem_p.at[b])
      return 0
    lax.fori_loop(0, ROWS_PER_CHUNK, row_copy, 0)

  issue(0, 0)

  def zero_body(k, _):
    hist[pl.ds(k * LANES, LANES)] = zeros
    return 0
  lax.fori_loop(0, (LANES * BINS_PAD) // LANES, zero_body, 0)

  def chunk_body(c, _):
    b = c % 2
    boff = b * CHUNK

    @pl.when(c + 1 < N_CHUNKS)
    def _():
      issue(c + 1, 1 - b)

    def row_wait(r, _):
      dst = pl.ds(boff + r * IMG_W, IMG_W)
      pltpu.make_async_copy(gt_hbm.at[img, row_base, :],
                            gt_buf.at[dst], sem_g.at[b]).wait()
      pltpu.make_async_copy(pre_hbm.at[img, row_base, :],
                            pre_buf.at[dst], sem_p.at[b]).wait()
      return 0
    lax.fori_loop(0, ROWS_PER_CHUNK, row_wait, 0)

    @plsc.parallel_loop(0, VECS_PER_CHUNK, unroll=16)
    def vec_body(i):
      g = gt_buf[pl.ds(boff + i * LANES, LANES)]
      p = pre_buf[pl.ds(boff + i * LANES, LANES)]
      idx = (g * NUM_CLASS + p) * LANES + lane
      plsc.addupdate_scatter(hist, [idx], ones)
    return 0
  lax.fori_loop(0, N_CHUNKS, chunk_body, 0)

  def scan_body(bb, _):
    v = hist[pl.ds(bb * LANES, LANES)]
    hist[pl.ds(bb * LANES, LANES)] = plsc.cumsum(v)
    return 0
  lax.fori_loop(0, BINS_PAD, scan_body, 0)

  def col_body(cc, _):
    idx = (cc * LANES + lane) * LANES + (LANES - 1)
    hist1d[pl.ds(cc * LANES, LANES)] = plsc.load_gather(hist, [idx])
    return 0
  lax.fori_loop(0, BINS_PAD // LANES, col_body, 0)

  pltpu.sync_copy(hist1d, out_hbm.at[wid])


@jax.jit
def _confusion(gt_img, pre_img):
  mesh = plsc.VectorSubcoreMesh(core_axis_name="c", subcore_axis_name="s")
  partials = pl.kernel(
      _sc_body,
      out_type=jax.ShapeDtypeStruct((NW, BINS_PAD), jnp.float32),
      mesh=mesh,
      compiler_params=pltpu.CompilerParams(needs_layout_passes=False),
      scratch_types=[
          pltpu.VMEM((2 * CHUNK,), jnp.int32),
          pltpu.VMEM((2 * CHUNK,), jnp.int32),
          pltpu.VMEM((LANES * BINS_PAD,), jnp.float32),
          pltpu.VMEM((BINS_PAD,), jnp.float32),
          pltpu.SemaphoreType.DMA((2,)),
          pltpu.SemaphoreType.DMA((2,)),
      ],
  )(gt_img, pre_img)
  return partials.sum(axis=0)[:NBINS].reshape(NUM_CLASS, NUM_CLASS)


def kernel(gt_image, pre_image):
  return _confusion(gt_image, pre_image)


# speedup vs baseline: 1.0706x; 1.0706x over previous
import functools

import jax
import jax.numpy as jnp
from jax import lax
from jax.experimental import pallas as pl
from jax.experimental.pallas import tpu as pltpu
from jax.experimental.pallas import tpu_sc as plsc

NUM_CLASS = 19
NBINS = NUM_CLASS * NUM_CLASS
BINS_PAD = 368
LANES = 16

N_TOTAL = 16 * 512 * 512
NC = 2
NS = 16
NW = NC * NS
IMG_H = 512
IMG_W = 512
ROWS_PER_W = 256
ROWS_PER_CHUNK = 32
CHUNK = ROWS_PER_CHUNK * IMG_W
N_CHUNKS = ROWS_PER_W // ROWS_PER_CHUNK
VECS_PER_CHUNK = CHUNK // LANES
VECS_PER_ROW = IMG_W // LANES


def _sc_body(gt_hbm, pre_hbm, out_hbm, gt_buf, pre_buf, hist, hist1d,
             sem_g, sem_p):
  wid = lax.axis_index("s") * NC + lax.axis_index("c")
  img = wid // 2
  row_base = (wid % 2) * ROWS_PER_W

  lane = jnp.arange(LANES, dtype=jnp.int32)
  ones = jnp.ones((LANES,), jnp.float32)
  zeros = jnp.zeros((LANES,), jnp.float32)

  def issue(c, b):
    r0 = row_base + c * ROWS_PER_CHUNK
    dst = pl.ds(b * ROWS_PER_CHUNK, ROWS_PER_CHUNK)
    pltpu.async_copy(gt_hbm.at[img, pl.ds(r0, ROWS_PER_CHUNK), :],
                     gt_buf.at[dst, :], sem_g.at[b])
    pltpu.async_copy(pre_hbm.at[img, pl.ds(r0, ROWS_PER_CHUNK), :],
                     pre_buf.at[dst, :], sem_p.at[b])

  issue(0, 0)

  @plsc.parallel_loop(0, (LANES * BINS_PAD) // LANES, unroll=8)
  def zero_body(k):
    hist[pl.ds(k * LANES, LANES)] = zeros

  def chunk_body(c, _):
    b = c % 2
    boff = b * ROWS_PER_CHUNK

    @pl.when(c + 1 < N_CHUNKS)
    def _():
      issue(c + 1, 1 - b)

    dst = pl.ds(boff, ROWS_PER_CHUNK)
    pltpu.make_async_copy(gt_hbm.at[img, pl.ds(row_base, ROWS_PER_CHUNK), :],
                          gt_buf.at[dst, :], sem_g.at[b]).wait()
    pltpu.make_async_copy(pre_hbm.at[img, pl.ds(row_base, ROWS_PER_CHUNK), :],
                          pre_buf.at[dst, :], sem_p.at[b]).wait()

    @plsc.parallel_loop(0, VECS_PER_CHUNK, unroll=16)
    def vec_body(i):
      rr = boff + (i // VECS_PER_ROW)
      cc = (i % VECS_PER_ROW) * LANES
      g = gt_buf[rr, pl.ds(cc, LANES)]
      p = pre_buf[rr, pl.ds(cc, LANES)]
      idx = (g * NUM_CLASS + p) * LANES + lane
      plsc.addupdate_scatter(hist, [idx], ones)
    return 0
  lax.fori_loop(0, N_CHUNKS, chunk_body, 0)

  @plsc.parallel_loop(0, BINS_PAD, unroll=8)
  def scan_body(bb):
    v = hist[pl.ds(bb * LANES, LANES)]
    hist[pl.ds(bb * LANES, LANES)] = plsc.cumsum(v)

  @plsc.parallel_loop(0, BINS_PAD // LANES, unroll=4)
  def col_body(cc):
    idx = (cc * LANES + lane) * LANES + (LANES - 1)
    hist1d[pl.ds(cc * LANES, LANES)] = plsc.load_gather(hist, [idx])

  pltpu.sync_copy(hist1d, out_hbm.at[wid])


@jax.jit
def _confusion(gt_img, pre_img):
  mesh = plsc.VectorSubcoreMesh(core_axis_name="c", subcore_axis_name="s")
  partials = pl.kernel(
      _sc_body,
      out_type=jax.ShapeDtypeStruct((NW, BINS_PAD), jnp.float32),
      mesh=mesh,
      compiler_params=pltpu.CompilerParams(needs_layout_passes=False),
      scratch_types=[
          pltpu.VMEM((2 * ROWS_PER_CHUNK, IMG_W), jnp.int32),
          pltpu.VMEM((2 * ROWS_PER_CHUNK, IMG_W), jnp.int32),
          pltpu.VMEM((LANES * BINS_PAD,), jnp.float32),
          pltpu.VMEM((BINS_PAD,), jnp.float32),
          pltpu.SemaphoreType.DMA((2,)),
          pltpu.SemaphoreType.DMA((2,)),
      ],
  )(gt_img, pre_img)
  return partials.sum(axis=0)[:NBINS].reshape(NUM_CLASS, NUM_CLASS)


def kernel(gt_image, pre_image):
  return _confusion(gt_image, pre_image)
